# baseline (speedup 1.0000x reference)
import jax
import jax.numpy as jnp
from jax import lax
from jax.experimental import pallas as pl
from jax.experimental.pallas import tpu as pltpu
from jax.experimental.pallas import tpu_sc as plsc

VOCAB = 1000000
SEQ_LEN = 200
EMBED_DIM = 64
BATCH = 4096
PAD_DIM = 128

N_ROWS = BATCH * SEQ_LEN
NUM_WORKERS = 32
ROWS_PER_WORKER = N_ROWS // NUM_WORKERS
BATCH_PER_CHUNK = 2
CHUNK = BATCH_PER_CHUNK * SEQ_LEN
NUM_CHUNKS = ROWS_PER_WORKER // CHUNK
IDX_STRIDE = 512
GATHER_BATCH = 128
FULL_GATHERS = CHUNK // GATHER_BATCH
TAIL = CHUNK - FULL_GATHERS * GATHER_BATCH


def _issue_gathers(token_hbm, idx_v, ibase, rows_ref, sem):
    sizes = [GATHER_BATCH] * FULL_GATHERS + ([TAIL] if TAIL else [])
    off = 0
    for n in sizes:
        pltpu.async_copy(
            token_hbm.at[idx_v.at[pl.ds(ibase + off, n)]],
            rows_ref.at[pl.ds(off, n)],
            sem,
            add=True,
        )
        off += n


def _body(inputs_hbm, token_hbm, pos_hbm, out_hbm,
          pos_sh, idx_v, rows_v, sem_g0, sem_g1, sem_o0, sem_o1):
    sid = lax.axis_index("s")
    wid = sid * 2 + lax.axis_index("c")
    base = wid * ROWS_PER_WORKER
    batch_base = wid * (ROWS_PER_WORKER // SEQ_LEN)
    sems_g = (sem_g0, sem_g1)
    sems_o = (sem_o0, sem_o1)

    @pl.when(sid == 0)
    def _():
        pltpu.sync_copy(pos_hbm, rows_v.at[0, pl.ds(0, SEQ_LEN)])
        pltpu.sync_copy(rows_v.at[0, pl.ds(0, SEQ_LEN)], pos_sh)

    plsc.subcore_barrier()

    def prep_and_fire(c, b):
        start = base + c * CHUNK
        for q in range(BATCH_PER_CHUNK):
            pltpu.sync_copy(
                pos_sh.at[slice(None), pl.ds(0, EMBED_DIM)],
                rows_v.at[b, pl.ds(q * SEQ_LEN, SEQ_LEN), pl.ds(0, EMBED_DIM)])
        pltpu.sync_copy(inputs_hbm.at[pl.ds(start, CHUNK)],
                        idx_v.at[pl.ds(b * IDX_STRIDE, CHUNK)])
        _issue_gathers(token_hbm, idx_v, b * IDX_STRIDE, rows_v.at[b],
                       sems_g[b])

    def wait_gathers(b):
        pltpu.make_async_copy(
            token_hbm.at[pl.ds(0, CHUNK)], rows_v.at[b], sems_g[b]).wait()

    def fire_writeback(c, b):
        brow = batch_base + c * BATCH_PER_CHUNK
        for r in range(BATCH_PER_CHUNK):
            pltpu.async_copy(
                rows_v.at[b, pl.ds(r * SEQ_LEN, SEQ_LEN)],
                out_hbm.at[brow + r],
                sems_o[b],
            )

    def wait_writeback(c, b):
        brow = batch_base + c * BATCH_PER_CHUNK
        for r in range(BATCH_PER_CHUNK):
            pltpu.make_async_copy(
                rows_v.at[b, pl.ds(r * SEQ_LEN, SEQ_LEN)],
                out_hbm.at[brow + r],
                sems_o[b],
            ).wait()

    prep_and_fire(0, 0)

    @pl.loop(1, NUM_CHUNKS)
    def _chunk(c):
        b = lax.rem(c, 2)

        @pl.when(b == 0)
        def _():
            @pl.when(c >= 2)
            def _():
                wait_writeback(c - 2, 0)
            prep_and_fire(c, 0)
            wait_gathers(1)
            fire_writeback(c - 1, 1)

        @pl.when(b == 1)
        def _():
            @pl.when(c >= 2)
            def _():
                wait_writeback(c - 2, 1)
            prep_and_fire(c, 1)
            wait_gathers(0)
            fire_writeback(c - 1, 0)

    last = NUM_CHUNKS - 1
    bl = last % 2
    wait_gathers(bl)
    fire_writeback(last, bl)
    wait_writeback(last - 1, 1 - bl)
    wait_writeback(last, bl)


@jax.jit
def _run(inputs_flat, token_padded, pos_padded):
    mesh = plsc.VectorSubcoreMesh(core_axis_name="c", subcore_axis_name="s")
    kern = pl.kernel(
        _body,
        out_type=jax.ShapeDtypeStruct((BATCH, SEQ_LEN, PAD_DIM), jnp.float32),
        mesh=mesh,
        scratch_types=[
            pltpu.VMEM_SHARED((SEQ_LEN, PAD_DIM), jnp.float32),
            pltpu.VMEM((2 * IDX_STRIDE,), jnp.int32),
            pltpu.VMEM((2, CHUNK, PAD_DIM), jnp.float32),
            pltpu.SemaphoreType.DMA,
            pltpu.SemaphoreType.DMA,
            pltpu.SemaphoreType.DMA,
            pltpu.SemaphoreType.DMA,
        ],
        compiler_params=pltpu.CompilerParams(use_tc_tiling_on_sc=True),
    )
    return kern(inputs_flat, token_padded, pos_padded)[:, :, :EMBED_DIM]


PAD_BLOCK = 8000


def _pad_body(x_ref, o_ref):
    x = x_ref[...]
    o_ref[...] = jnp.concatenate(
        [x, jnp.zeros((PAD_BLOCK, PAD_DIM - EMBED_DIM), jnp.float32)], axis=1)


def _pad_table(token_table):
    grid = VOCAB // PAD_BLOCK
    return pl.pallas_call(
        _pad_body,
        out_shape=jax.ShapeDtypeStruct((VOCAB, PAD_DIM), jnp.float32),
        grid=(grid,),
        in_specs=[pl.BlockSpec((PAD_BLOCK, EMBED_DIM), lambda i: (i, 0))],
        out_specs=pl.BlockSpec((PAD_BLOCK, PAD_DIM), lambda i: (i, 0)),
        compiler_params=pltpu.CompilerParams(
            dimension_semantics=("arbitrary",)),
    )(token_table)


def kernel(inputs, token_table, position_table):
    inputs_flat = inputs.reshape(-1).astype(jnp.int32)
    token_padded = _pad_table(token_table)
    pos_padded = jnp.pad(position_table, ((0, 0), (0, PAD_DIM - EMBED_DIM)))
    return _run(inputs_flat, token_padded, pos_padded)


# speedup vs baseline: 1.1284x; 1.1284x over previous
import jax
import jax.numpy as jnp
from jax import lax
from jax.experimental import pallas as pl
from jax.experimental.pallas import tpu as pltpu
from jax.experimental.pallas import tpu_sc as plsc

VOCAB = 1000000
SEQ_LEN = 200
EMBED_DIM = 64
BATCH = 4096
PAD_DIM = 128

N_ROWS = BATCH * SEQ_LEN
NUM_WORKERS = 32
ROWS_PER_WORKER = N_ROWS // NUM_WORKERS
BATCH_PER_CHUNK = 2
CHUNK = BATCH_PER_CHUNK * SEQ_LEN
NUM_CHUNKS = ROWS_PER_WORKER // CHUNK
IDX_STRIDE = 512
GATHER_BATCH = 128
FULL_GATHERS = CHUNK // GATHER_BATCH
TAIL = CHUNK - FULL_GATHERS * GATHER_BATCH


def _issue_gathers(token_hbm, idx_v, ibase, rows_ref, sem):
    sizes = [GATHER_BATCH] * FULL_GATHERS + ([TAIL] if TAIL else [])
    off = 0
    for n in sizes:
        pltpu.async_copy(
            token_hbm.at[idx_v.at[pl.ds(ibase + off, n)]],
            rows_ref.at[pl.ds(off, n)],
            sem,
            add=True,
        )
        off += n


def _body(inputs_hbm, token_hbm, pos_hbm, out_hbm,
          pos_sh, idx_v, rows_v, sem_g0, sem_g1, sem_o0, sem_o1):
    sid = lax.axis_index("s")
    wid = sid * 2 + lax.axis_index("c")
    base = wid * ROWS_PER_WORKER
    batch_base = wid * (ROWS_PER_WORKER // SEQ_LEN)
    sems_g = (sem_g0, sem_g1)
    sems_o = (sem_o0, sem_o1)

    @pl.when(sid == 0)
    def _():
        pltpu.sync_copy(pos_hbm, rows_v.at[0, pl.ds(0, SEQ_LEN)])
        pltpu.sync_copy(rows_v.at[0, pl.ds(0, SEQ_LEN)], pos_sh)

    plsc.subcore_barrier()

    def prep_and_fire(c, b):
        start = base + c * CHUNK
        for q in range(BATCH_PER_CHUNK):
            pltpu.sync_copy(
                pos_sh.at[slice(None), pl.ds(0, EMBED_DIM)],
                rows_v.at[b, pl.ds(q * SEQ_LEN, SEQ_LEN), pl.ds(0, EMBED_DIM)])
        pltpu.sync_copy(inputs_hbm.at[pl.ds(start, CHUNK)],
                        idx_v.at[pl.ds(b * IDX_STRIDE, CHUNK)])
        _issue_gathers(token_hbm, idx_v, b * IDX_STRIDE, rows_v.at[b],
                       sems_g[b])

    def wait_gathers(b):
        pltpu.make_async_copy(
            token_hbm.at[pl.ds(0, CHUNK)], rows_v.at[b], sems_g[b]).wait()

    def fire_writeback(c, b):
        brow = batch_base + c * BATCH_PER_CHUNK
        for r in range(BATCH_PER_CHUNK):
            pltpu.async_copy(
                rows_v.at[b, pl.ds(r * SEQ_LEN, SEQ_LEN)],
                out_hbm.at[brow + r],
                sems_o[b],
            )

    def wait_writeback(c, b):
        brow = batch_base + c * BATCH_PER_CHUNK
        for r in range(BATCH_PER_CHUNK):
            pltpu.make_async_copy(
                rows_v.at[b, pl.ds(r * SEQ_LEN, SEQ_LEN)],
                out_hbm.at[brow + r],
                sems_o[b],
            ).wait()

    prep_and_fire(0, 0)

    @pl.loop(1, NUM_CHUNKS)
    def _chunk(c):
        b = lax.rem(c, 2)

        @pl.when(b == 0)
        def _():
            @pl.when(c >= 2)
            def _():
                wait_writeback(c - 2, 0)
            prep_and_fire(c, 0)
            wait_gathers(1)
            fire_writeback(c - 1, 1)

        @pl.when(b == 1)
        def _():
            @pl.when(c >= 2)
            def _():
                wait_writeback(c - 2, 1)
            prep_and_fire(c, 1)
            wait_gathers(0)
            fire_writeback(c - 1, 0)

    last = NUM_CHUNKS - 1
    bl = last % 2
    wait_gathers(bl)
    fire_writeback(last, bl)
    wait_writeback(last - 1, 1 - bl)
    wait_writeback(last, bl)


@jax.jit
def _run(inputs_flat, token_padded, pos_padded):
    mesh = plsc.VectorSubcoreMesh(core_axis_name="c", subcore_axis_name="s")
    kern = pl.kernel(
        _body,
        out_type=jax.ShapeDtypeStruct((BATCH, SEQ_LEN, PAD_DIM), jnp.float32),
        mesh=mesh,
        scratch_types=[
            pltpu.VMEM_SHARED((SEQ_LEN, PAD_DIM), jnp.float32),
            pltpu.VMEM((2 * IDX_STRIDE,), jnp.int32),
            pltpu.VMEM((2, CHUNK, PAD_DIM), jnp.float32),
            pltpu.SemaphoreType.DMA,
            pltpu.SemaphoreType.DMA,
            pltpu.SemaphoreType.DMA,
            pltpu.SemaphoreType.DMA,
        ],
        compiler_params=pltpu.CompilerParams(use_tc_tiling_on_sc=True),
    )
    return kern(inputs_flat, token_padded, pos_padded)[:, :, :EMBED_DIM]


def kernel(inputs, token_table, position_table):
    inputs_flat = inputs.reshape(-1).astype(jnp.int32)
    token_padded = jnp.pad(token_table, ((0, 0), (0, PAD_DIM - EMBED_DIM)))
    pos_padded = jnp.pad(position_table, ((0, 0), (0, PAD_DIM - EMBED_DIM)))
    return _run(inputs_flat, token_padded, pos_padded)


# speedup vs baseline: 1.3162x; 1.1664x over previous
import jax
import jax.numpy as jnp
from jax import lax
from jax.experimental import pallas as pl
from jax.experimental.pallas import tpu as pltpu
from jax.experimental.pallas import tpu_sc as plsc

VOCAB = 1000000
SEQ_LEN = 200
EMBED_DIM = 64
BATCH = 4096
PAD_DIM = 128

N_ROWS = BATCH * SEQ_LEN
NUM_WORKERS = 32
ROWS_PER_WORKER = N_ROWS // NUM_WORKERS
BATCH_PER_CHUNK = 4
CHUNK = BATCH_PER_CHUNK * SEQ_LEN
NUM_CHUNKS = ROWS_PER_WORKER // CHUNK
GATHER_BATCH = 128
FULL_GATHERS = CHUNK // GATHER_BATCH
TAIL = CHUNK - FULL_GATHERS * GATHER_BATCH


def _issue_gathers(token_hbm, idx_ref, rows_ref, sem):
    sizes = [GATHER_BATCH] * FULL_GATHERS + ([TAIL] if TAIL else [])
    off = 0
    for n in sizes:
        pltpu.async_copy(
            token_hbm.at[idx_ref.at[pl.ds(off, n)]],
            rows_ref.at[pl.ds(off, n)],
            sem,
            add=True,
        )
        off += n


def _body(inputs_hbm, token_hbm, pos_hbm, out_hbm,
          pos_sh, idx_v, rows_v, sem_g0, sem_g1, sem_o0, sem_o1):
    sid = lax.axis_index("s")
    wid = sid * 2 + lax.axis_index("c")
    base = wid * ROWS_PER_WORKER
    batch_base = wid * (ROWS_PER_WORKER // SEQ_LEN)
    sems_g = (sem_g0, sem_g1)
    sems_o = (sem_o0, sem_o1)

    @pl.when(sid == 0)
    def _():
        pltpu.sync_copy(pos_hbm, rows_v.at[0, pl.ds(0, SEQ_LEN)])
        pltpu.sync_copy(rows_v.at[0, pl.ds(0, SEQ_LEN)], pos_sh)

    plsc.subcore_barrier()

    def prep_and_fire(c, b):
        start = base + c * CHUNK
        for q in range(BATCH_PER_CHUNK):
            pltpu.sync_copy(pos_sh, rows_v.at[b, pl.ds(q * SEQ_LEN, SEQ_LEN)])
        pltpu.sync_copy(inputs_hbm.at[pl.ds(start, CHUNK)], idx_v.at[b])
        _issue_gathers(token_hbm, idx_v.at[b], rows_v.at[b], sems_g[b])

    def wait_gathers(b):
        pltpu.make_async_copy(
            token_hbm.at[pl.ds(0, CHUNK)], rows_v.at[b], sems_g[b]).wait()

    def fire_writeback(c, b):
        brow = batch_base + c * BATCH_PER_CHUNK
        for r in range(BATCH_PER_CHUNK):
            pltpu.async_copy(
                rows_v.at[b, pl.ds(r * SEQ_LEN, SEQ_LEN)],
                out_hbm.at[brow + r, slice(None), pl.ds(0, EMBED_DIM)],
                sems_o[b],
            )

    def wait_writeback(c, b):
        brow = batch_base + c * BATCH_PER_CHUNK
        for r in range(BATCH_PER_CHUNK):
            pltpu.make_async_copy(
                rows_v.at[b, pl.ds(r * SEQ_LEN, SEQ_LEN)],
                out_hbm.at[brow + r, slice(None), pl.ds(0, EMBED_DIM)],
                sems_o[b],
            ).wait()

    prep_and_fire(0, 0)

    @pl.loop(1, NUM_CHUNKS)
    def _chunk(c):
        b = lax.rem(c, 2)

        @pl.when(b == 0)
        def _():
            @pl.when(c >= 2)
            def _():
                wait_writeback(c - 2, 0)
            prep_and_fire(c, 0)
            wait_gathers(1)
            fire_writeback(c - 1, 1)

        @pl.when(b == 1)
        def _():
            @pl.when(c >= 2)
            def _():
                wait_writeback(c - 2, 1)
            prep_and_fire(c, 1)
            wait_gathers(0)
            fire_writeback(c - 1, 0)

    last = NUM_CHUNKS - 1
    bl = last % 2
    wait_gathers(bl)
    fire_writeback(last, bl)
    wait_writeback(last - 1, 1 - bl)
    wait_writeback(last, bl)


@jax.jit
def _run(inputs2, token_2m, position_table):
    mesh = plsc.VectorSubcoreMesh(core_axis_name="c", subcore_axis_name="s")
    kern = pl.kernel(
        _body,
        out_type=jax.ShapeDtypeStruct((BATCH, SEQ_LEN, PAD_DIM), jnp.float32),
        mesh=mesh,
        scratch_types=[
            pltpu.VMEM_SHARED((SEQ_LEN, EMBED_DIM), jnp.float32),
            pltpu.VMEM((2, CHUNK), jnp.int32),
            pltpu.VMEM((2, CHUNK, EMBED_DIM), jnp.float32),
            pltpu.SemaphoreType.DMA,
            pltpu.SemaphoreType.DMA,
            pltpu.SemaphoreType.DMA,
            pltpu.SemaphoreType.DMA,
        ],
        compiler_params=pltpu.CompilerParams(use_tc_tiling_on_sc=False),
    )
    return kern(inputs2, token_2m, position_table)[:, :, :EMBED_DIM]


def kernel(inputs, token_table, position_table):
    inputs2 = inputs.reshape(-1).astype(jnp.int32) * 2
    token_2m = jnp.pad(
        token_table, ((0, 0), (0, PAD_DIM - EMBED_DIM))
    ).reshape(2 * VOCAB, EMBED_DIM)
    return _run(inputs2, token_2m, position_table)


# speedup vs baseline: 1.4019x; 1.0651x over previous
import jax
import jax.numpy as jnp
from jax import lax
from jax.experimental import pallas as pl
from jax.experimental.pallas import tpu as pltpu
from jax.experimental.pallas import tpu_sc as plsc

VOCAB = 1000000
SEQ_LEN = 200
EMBED_DIM = 64
BATCH = 4096
PAD_DIM = 128

N_ROWS = BATCH * SEQ_LEN
NUM_WORKERS = 32
ROWS_PER_WORKER = N_ROWS // NUM_WORKERS
BATCH_PER_CHUNK = 4
CHUNK = BATCH_PER_CHUNK * SEQ_LEN
NUM_CHUNKS = ROWS_PER_WORKER // CHUNK
GATHER_BATCH = 128
FULL_GATHERS = CHUNK // GATHER_BATCH
TAIL = CHUNK - FULL_GATHERS * GATHER_BATCH


def _issue_gathers(token_hbm, idx_ref, rows_ref, sem):
    sizes = [GATHER_BATCH] * FULL_GATHERS + ([TAIL] if TAIL else [])
    off = 0
    for n in sizes:
        pltpu.async_copy(
            token_hbm.at[idx_ref.at[pl.ds(off, n)]],
            rows_ref.at[pl.ds(off, n)],
            sem,
            add=True,
        )
        off += n


def _body(inputs_hbm, token_hbm, pos_hbm, out_hbm,
          pos_sh, idx_v, rows_v, sem_g0, sem_g1, sem_o0, sem_o1):
    sid = lax.axis_index("s")
    wid = sid * 2 + lax.axis_index("c")
    base = wid * ROWS_PER_WORKER
    batch_base = wid * (ROWS_PER_WORKER // SEQ_LEN)
    sems_g = (sem_g0, sem_g1)
    sems_o = (sem_o0, sem_o1)

    @pl.when(sid == 0)
    def _():
        pltpu.sync_copy(pos_hbm, rows_v.at[0, pl.ds(0, SEQ_LEN)])
        pltpu.sync_copy(rows_v.at[0, pl.ds(0, SEQ_LEN)], pos_sh)

    plsc.subcore_barrier()

    def prep_and_fire(c, b):
        start = base + c * CHUNK
        for q in range(BATCH_PER_CHUNK):
            pltpu.sync_copy(pos_sh, rows_v.at[b, pl.ds(q * SEQ_LEN, SEQ_LEN)])
        pltpu.sync_copy(inputs_hbm.at[pl.ds(start, CHUNK)], idx_v.at[b])
        _issue_gathers(token_hbm, idx_v.at[b], rows_v.at[b], sems_g[b])

    def wait_gathers(b):
        pltpu.make_async_copy(
            token_hbm.at[pl.ds(0, CHUNK)], rows_v.at[b], sems_g[b]).wait()

    def fire_writeback(c, b):
        brow = batch_base + c * BATCH_PER_CHUNK
        for r in range(BATCH_PER_CHUNK):
            pltpu.async_copy(
                rows_v.at[b, pl.ds(r * SEQ_LEN, SEQ_LEN)],
                out_hbm.at[brow + r, slice(None), pl.ds(0, EMBED_DIM)],
                sems_o[b],
            )

    def wait_writeback(c, b):
        brow = batch_base + c * BATCH_PER_CHUNK
        for r in range(BATCH_PER_CHUNK):
            pltpu.make_async_copy(
                rows_v.at[b, pl.ds(r * SEQ_LEN, SEQ_LEN)],
                out_hbm.at[brow + r, slice(None), pl.ds(0, EMBED_DIM)],
                sems_o[b],
            ).wait()

    prep_and_fire(0, 0)

    @pl.loop(1, NUM_CHUNKS)
    def _chunk(c):
        b = lax.rem(c, 2)

        @pl.when(b == 0)
        def _():
            @pl.when(c >= 2)
            def _():
                wait_writeback(c - 2, 0)
            prep_and_fire(c, 0)
            wait_gathers(1)
            fire_writeback(c - 1, 1)

        @pl.when(b == 1)
        def _():
            @pl.when(c >= 2)
            def _():
                wait_writeback(c - 2, 1)
            prep_and_fire(c, 1)
            wait_gathers(0)
            fire_writeback(c - 1, 0)

    last = NUM_CHUNKS - 1
    bl = last % 2
    wait_gathers(bl)
    fire_writeback(last, bl)
    wait_writeback(last - 1, 1 - bl)
    wait_writeback(last, bl)


@jax.jit
def _run(inputs2, token_2m, position_table):
    mesh = plsc.VectorSubcoreMesh(core_axis_name="c", subcore_axis_name="s")
    kern = pl.kernel(
        _body,
        out_type=jax.ShapeDtypeStruct((BATCH, SEQ_LEN, PAD_DIM), jnp.float32),
        mesh=mesh,
        scratch_types=[
            pltpu.VMEM_SHARED((SEQ_LEN, EMBED_DIM), jnp.float32),
            pltpu.VMEM((2, CHUNK), jnp.int32),
            pltpu.VMEM((2, CHUNK, EMBED_DIM), jnp.float32),
            pltpu.SemaphoreType.DMA,
            pltpu.SemaphoreType.DMA,
            pltpu.SemaphoreType.DMA,
            pltpu.SemaphoreType.DMA,
        ],
        compiler_params=pltpu.CompilerParams(use_tc_tiling_on_sc=False),
    )
    return kern(inputs2, token_2m, position_table)[:, :, :EMBED_DIM]


TP_BLK = 2048
TP_GRID = -(-VOCAB // TP_BLK)


def _tp_body(xT_ref, o_ref):
    xt = xT_ref[...].T
    o_ref[...] = jnp.concatenate(
        [xt, jnp.zeros((TP_BLK, EMBED_DIM), jnp.float32)], axis=1)


def _transpose_pad(tokT):
    return pl.pallas_call(
        _tp_body,
        out_shape=jax.ShapeDtypeStruct((VOCAB, PAD_DIM), jnp.float32),
        grid=(TP_GRID,),
        in_specs=[pl.BlockSpec((EMBED_DIM, TP_BLK), lambda i: (0, i))],
        out_specs=pl.BlockSpec((TP_BLK, PAD_DIM), lambda i: (i, 0)),
        compiler_params=pltpu.CompilerParams(
            dimension_semantics=("arbitrary",)),
    )(tokT)


def kernel(inputs, token_table, position_table):
    inputs2 = inputs.reshape(-1).astype(jnp.int32) * 2
    token_2m = _transpose_pad(token_table.T).reshape(2 * VOCAB, EMBED_DIM)
    return _run(inputs2, token_2m, position_table)


# speedup vs baseline: 1.8270x; 1.3032x over previous
import jax
import jax.numpy as jnp
from jax import lax
from jax.experimental import pallas as pl
from jax.experimental.pallas import tpu as pltpu
from jax.experimental.pallas import tpu_sc as plsc

VOCAB = 1000000
SEQ_LEN = 200
EMBED_DIM = 64
BATCH = 4096
PAD_DIM = 128

N_ROWS = BATCH * SEQ_LEN
NUM_WORKERS = 32
ROWS_PER_WORKER = N_ROWS // NUM_WORKERS
BATCH_PER_CHUNK = 4
CHUNK = BATCH_PER_CHUNK * SEQ_LEN
NUM_CHUNKS = ROWS_PER_WORKER // CHUNK
GATHER_BATCH = 128
FULL_GATHERS = CHUNK // GATHER_BATCH
TAIL = CHUNK - FULL_GATHERS * GATHER_BATCH


def _issue_gathers(token_hbm, idx_ref, rows_ref, sem):
    sizes = [GATHER_BATCH] * FULL_GATHERS + ([TAIL] if TAIL else [])
    off = 0
    for n in sizes:
        pltpu.async_copy(
            token_hbm.at[idx_ref.at[pl.ds(off, n)]],
            rows_ref.at[pl.ds(off, n)],
            sem,
            add=True,
        )
        off += n


def _body(inputs_hbm, token_hbm, pos_hbm, out_hbm,
          pos_sh, idx_v, rows_v, sem_g0, sem_g1, sem_o0, sem_o1):
    sid = lax.axis_index("s")
    wid = sid * 2 + lax.axis_index("c")
    base = wid * ROWS_PER_WORKER
    batch_base = wid * (ROWS_PER_WORKER // SEQ_LEN)
    sems_g = (sem_g0, sem_g1)
    sems_o = (sem_o0, sem_o1)

    @pl.when(sid == 0)
    def _():
        pltpu.sync_copy(pos_hbm, rows_v.at[0, pl.ds(0, SEQ_LEN)])
        pltpu.sync_copy(rows_v.at[0, pl.ds(0, SEQ_LEN)], pos_sh)

    plsc.subcore_barrier()

    def prep_and_fire(c, b):
        start = base + c * CHUNK
        for q in range(BATCH_PER_CHUNK):
            pltpu.sync_copy(pos_sh, rows_v.at[b, pl.ds(q * SEQ_LEN, SEQ_LEN)])
        pltpu.sync_copy(inputs_hbm.at[pl.ds(start, CHUNK)], idx_v.at[b])
        _issue_gathers(token_hbm, idx_v.at[b], rows_v.at[b], sems_g[b])

    def wait_gathers(b):
        pltpu.make_async_copy(
            token_hbm.at[pl.ds(0, CHUNK)], rows_v.at[b], sems_g[b]).wait()

    def fire_writeback(c, b):
        brow = batch_base + c * BATCH_PER_CHUNK
        for r in range(BATCH_PER_CHUNK):
            pltpu.async_copy(
                rows_v.at[b, pl.ds(r * SEQ_LEN, SEQ_LEN)],
                out_hbm.at[brow + r, slice(None), pl.ds(0, EMBED_DIM)],
                sems_o[b],
            )

    def wait_writeback(c, b):
        brow = batch_base + c * BATCH_PER_CHUNK
        for r in range(BATCH_PER_CHUNK):
            pltpu.make_async_copy(
                rows_v.at[b, pl.ds(r * SEQ_LEN, SEQ_LEN)],
                out_hbm.at[brow + r, slice(None), pl.ds(0, EMBED_DIM)],
                sems_o[b],
            ).wait()

    prep_and_fire(0, 0)

    @pl.loop(1, NUM_CHUNKS)
    def _chunk(c):
        b = lax.rem(c, 2)

        @pl.when(b == 0)
        def _():
            @pl.when(c >= 2)
            def _():
                wait_writeback(c - 2, 0)
            prep_and_fire(c, 0)
            wait_gathers(1)
            fire_writeback(c - 1, 1)

        @pl.when(b == 1)
        def _():
            @pl.when(c >= 2)
            def _():
                wait_writeback(c - 2, 1)
            prep_and_fire(c, 1)
            wait_gathers(0)
            fire_writeback(c - 1, 0)

    last = NUM_CHUNKS - 1
    bl = last % 2
    wait_gathers(bl)
    fire_writeback(last, bl)
    wait_writeback(last - 1, 1 - bl)
    wait_writeback(last, bl)


@jax.jit
def _run(inputs2, token_2m, position_table):
    mesh = plsc.VectorSubcoreMesh(core_axis_name="c", subcore_axis_name="s")
    kern = pl.kernel(
        _body,
        out_type=jax.ShapeDtypeStruct((BATCH, SEQ_LEN, PAD_DIM), jnp.float32),
        mesh=mesh,
        scratch_types=[
            pltpu.VMEM_SHARED((SEQ_LEN, EMBED_DIM), jnp.float32),
            pltpu.VMEM((2, CHUNK), jnp.int32),
            pltpu.VMEM((2, CHUNK, EMBED_DIM), jnp.float32),
            pltpu.SemaphoreType.DMA,
            pltpu.SemaphoreType.DMA,
            pltpu.SemaphoreType.DMA,
            pltpu.SemaphoreType.DMA,
        ],
        compiler_params=pltpu.CompilerParams(use_tc_tiling_on_sc=False),
    )
    return kern(inputs2, token_2m, position_table)[:, :, :EMBED_DIM]


TP_BLK = 8192
TP_GRID = -(-VOCAB // TP_BLK)


def _tp_body(xT_ref, o_ref):
    xt = xT_ref[...].T
    o_ref[...] = jnp.concatenate(
        [xt, jnp.zeros((TP_BLK, EMBED_DIM), jnp.float32)], axis=1)


def _transpose_pad(tokT):
    return pl.pallas_call(
        _tp_body,
        out_shape=jax.ShapeDtypeStruct((VOCAB, PAD_DIM), jnp.float32),
        grid=(TP_GRID,),
        in_specs=[pl.BlockSpec((EMBED_DIM, TP_BLK), lambda i: (0, i))],
        out_specs=pl.BlockSpec((TP_BLK, PAD_DIM), lambda i: (i, 0)),
        compiler_params=pltpu.CompilerParams(
            dimension_semantics=("arbitrary",)),
    )(tokT)


def kernel(inputs, token_table, position_table):
    inputs2 = inputs.reshape(-1).astype(jnp.int32) * 2
    token_2m = _transpose_pad(token_table.T).reshape(2 * VOCAB, EMBED_DIM)
    return _run(inputs2, token_2m, position_table)


# speedup vs baseline: 1.8837x; 1.0310x over previous
import jax
import jax.numpy as jnp
from jax import lax
from jax.experimental import pallas as pl
from jax.experimental.pallas import tpu as pltpu
from jax.experimental.pallas import tpu_sc as plsc

VOCAB = 1000000
SEQ_LEN = 200
EMBED_DIM = 64
BATCH = 4096
PAD_DIM = 128

N_ROWS = BATCH * SEQ_LEN
NUM_WORKERS = 32
ROWS_PER_WORKER = N_ROWS // NUM_WORKERS
BATCH_PER_CHUNK = 4
CHUNK = BATCH_PER_CHUNK * SEQ_LEN
NUM_CHUNKS = ROWS_PER_WORKER // CHUNK
GATHER_BATCH = 128
FULL_GATHERS = CHUNK // GATHER_BATCH
TAIL = CHUNK - FULL_GATHERS * GATHER_BATCH


def _issue_gathers(token_hbm, idx_ref, rows_ref, sem):
    sizes = [GATHER_BATCH] * FULL_GATHERS + ([TAIL] if TAIL else [])
    off = 0
    for n in sizes:
        pltpu.async_copy(
            token_hbm.at[idx_ref.at[pl.ds(off, n)]],
            rows_ref.at[pl.ds(off, n)],
            sem,
            add=True,
        )
        off += n


def _body(inputs_hbm, token_hbm, pos_hbm, out_hbm,
          pos_sh, idx_v, rows_v, sem_g0, sem_g1, sem_o0, sem_o1):
    sid = lax.axis_index("s")
    wid = sid * 2 + lax.axis_index("c")
    base = wid * ROWS_PER_WORKER
    batch_base = wid * (ROWS_PER_WORKER // SEQ_LEN)
    sems_g = (sem_g0, sem_g1)
    sems_o = (sem_o0, sem_o1)

    @pl.when(sid == 0)
    def _():
        pltpu.sync_copy(pos_hbm, rows_v.at[0, pl.ds(0, SEQ_LEN)])
        pltpu.sync_copy(rows_v.at[0, pl.ds(0, SEQ_LEN)], pos_sh)

    plsc.subcore_barrier()

    def prep_and_fire(c, b):
        start = base + c * CHUNK
        for q in range(BATCH_PER_CHUNK):
            pltpu.sync_copy(pos_sh, rows_v.at[b, pl.ds(q * SEQ_LEN, SEQ_LEN)])
        pltpu.sync_copy(inputs_hbm.at[pl.ds(start, CHUNK)], idx_v.at[b])
        _issue_gathers(token_hbm, idx_v.at[b], rows_v.at[b], sems_g[b])

    def wait_gathers(b):
        pltpu.make_async_copy(
            token_hbm.at[pl.ds(0, CHUNK)], rows_v.at[b], sems_g[b]).wait()

    def fire_writeback(c, b):
        brow = batch_base + c * BATCH_PER_CHUNK
        for r in range(BATCH_PER_CHUNK):
            pltpu.async_copy(
                rows_v.at[b, pl.ds(r * SEQ_LEN, SEQ_LEN)],
                out_hbm.at[brow + r, slice(None), pl.ds(0, EMBED_DIM)],
                sems_o[b],
            )

    def wait_writeback(c, b):
        brow = batch_base + c * BATCH_PER_CHUNK
        for r in range(BATCH_PER_CHUNK):
            pltpu.make_async_copy(
                rows_v.at[b, pl.ds(r * SEQ_LEN, SEQ_LEN)],
                out_hbm.at[brow + r, slice(None), pl.ds(0, EMBED_DIM)],
                sems_o[b],
            ).wait()

    prep_and_fire(0, 0)

    @pl.loop(1, NUM_CHUNKS)
    def _chunk(c):
        b = lax.rem(c, 2)

        @pl.when(b == 0)
        def _():
            @pl.when(c >= 2)
            def _():
                wait_writeback(c - 2, 0)
            prep_and_fire(c, 0)
            wait_gathers(1)
            fire_writeback(c - 1, 1)

        @pl.when(b == 1)
        def _():
            @pl.when(c >= 2)
            def _():
                wait_writeback(c - 2, 1)
            prep_and_fire(c, 1)
            wait_gathers(0)
            fire_writeback(c - 1, 0)

    last = NUM_CHUNKS - 1
    bl = last % 2
    wait_gathers(bl)
    fire_writeback(last, bl)
    wait_writeback(last - 1, 1 - bl)
    wait_writeback(last, bl)


@jax.jit
def _run(inputs2, token_2m, position_table):
    mesh = plsc.VectorSubcoreMesh(core_axis_name="c", subcore_axis_name="s")
    kern = pl.kernel(
        _body,
        out_type=jax.ShapeDtypeStruct((BATCH, SEQ_LEN, PAD_DIM), jnp.float32),
        mesh=mesh,
        scratch_types=[
            pltpu.VMEM_SHARED((SEQ_LEN, EMBED_DIM), jnp.float32),
            pltpu.VMEM((2, CHUNK), jnp.int32),
            pltpu.VMEM((2, CHUNK, EMBED_DIM), jnp.float32),
            pltpu.SemaphoreType.DMA,
            pltpu.SemaphoreType.DMA,
            pltpu.SemaphoreType.DMA,
            pltpu.SemaphoreType.DMA,
        ],
        compiler_params=pltpu.CompilerParams(use_tc_tiling_on_sc=False),
    )
    return kern(inputs2, token_2m, position_table)[:, :, :EMBED_DIM]


TP_BLK = 16384
TP_GRID = -(-VOCAB // TP_BLK)


def _tp_body(xT_ref, o_ref):
    xt = xT_ref[...].T
    o_ref[...] = jnp.concatenate(
        [xt, jnp.zeros((TP_BLK, EMBED_DIM), jnp.float32)], axis=1)


def _transpose_pad(tokT):
    return pl.pallas_call(
        _tp_body,
        out_shape=jax.ShapeDtypeStruct((VOCAB, PAD_DIM), jnp.float32),
        grid=(TP_GRID,),
        in_specs=[pl.BlockSpec((EMBED_DIM, TP_BLK), lambda i: (0, i))],
        out_specs=pl.BlockSpec((TP_BLK, PAD_DIM), lambda i: (i, 0)),
        compiler_params=pltpu.CompilerParams(
            dimension_semantics=("arbitrary",)),
    )(tokT)


def kernel(inputs, token_table, position_table):
    inputs2 = inputs.reshape(-1).astype(jnp.int32) * 2
    token_2m = _transpose_pad(token_table.T).reshape(2 * VOCAB, EMBED_DIM)
    return _run(inputs2, token_2m, position_table)
